# baseline (speedup 1.0000x reference)
import functools

import jax
import jax.numpy as jnp
from jax import lax
from jax.experimental import pallas as pl
from jax.experimental.pallas import tpu as pltpu
from jax.experimental.pallas import tpu_sc as plsc

_DIM = 32
_L = 16
_CHUNK = 128
_RING = 8
_PF = 7
_PITCH = _CHUNK + 1
_RUR = 4


def _mod1(v):
    r = lax.rem(v, 1.0)
    return jnp.where(r < 0.0, r + 1.0, r)


@functools.lru_cache(maxsize=None)
def _make_sc_kernel(n_j, n_i, nw, nc):
    ipw = n_i // nw
    tcw = ipw // _CHUNK
    n_chunks = n_j * tcw
    n_groups = n_chunks // _RING
    n_ktiles = _DIM // 8
    n_tc = n_i // _CHUNK
    out_flat = jax.ShapeDtypeStruct((n_j * n_ktiles * n_tc * 8, _CHUNK),
                                    jnp.float32)
    mesh = plsc.VectorSubcoreMesh(core_axis_name="c", subcore_axis_name="s")

    @functools.partial(
        pl.kernel,
        mesh=mesh,
        out_type=(out_flat,) * 4,
        compiler_params=pltpu.CompilerParams(
            use_tc_tiling_on_sc=False, needs_layout_passes=False),
        scratch_types=(
            [pltpu.VMEM((n_chunks, _CHUNK), jnp.int32)]
            + [pltpu.VMEM((_CHUNK, _DIM), jnp.float32)] * (2 * _RING)
            + [pltpu.VMEM((_DIM, _PITCH), jnp.float32)] * 8
            + [pltpu.SemaphoreType.DMA] * (_RING + 2)
        ),
    )
    def sc_kernel(xq, bt, et, ob, oe, od, oa, idx_v, *bufs):
        brefs = bufs[:_RING]
        erefs = bufs[_RING:2 * _RING]
        sb0, se0, sd0, sa0, sb1, se1, sd1, sa1 = bufs[2 * _RING:
                                                      2 * _RING + 8]
        gsem = bufs[2 * _RING + 8:3 * _RING + 8]
        wsem = bufs[3 * _RING + 8:]
        stag = ((sb0, se0, sd0, sa0), (sb1, se1, sd1, sa1))
        c_ax = lax.axis_index("c")
        s_ax = lax.axis_index("s")
        wid = s_ax * nc + c_ax
        pltpu.sync_copy(xq.at[wid], idx_v)

        kvs = [lax.iota(jnp.int32, 16) + (h * _L) for h in range(_DIM // _L)]

        def issue_gather(cidx, slot):
            pltpu.async_copy(bt.at[idx_v.at[cidx]], brefs[slot], gsem[slot])
            pltpu.async_copy(et.at[idx_v.at[cidx]], erefs[slot], gsem[slot])

        for u in range(_PF):
            issue_gather(u, u)

        def group(gi, carry):
            for u in range(_RING):
                cc = gi * _RING + u
                s = u & 1
                sbs, ses, sds, sas = stag[s]
                bg = brefs[u]
                eg = erefs[u]

                pslot = (u + _PF) % _RING
                if u == 0:
                    issue_gather(cc + _PF, pslot)
                else:
                    @pl.when(gi < n_groups - 1)
                    def _pref(_cc=cc, _ps=pslot):
                        issue_gather(_cc + _PF, _ps)

                pltpu.make_async_copy(
                    bt.at[idx_v.at[cc]], bg, gsem[u]).wait()
                pltpu.make_async_copy(
                    et.at[idx_v.at[cc]], eg, gsem[u]).wait()

                def drain(_s=s):
                    for o_ref in stag[_s]:
                        for tr in range(n_ktiles):
                            pltpu.make_async_copy(
                                o_ref.at[pl.ds(tr * 8, 8), pl.ds(0, _CHUNK)],
                                ob.at[pl.ds(0, 8)], wsem[_s]).wait()
                if u >= 2:
                    drain()
                else:
                    @pl.when(gi >= 1)
                    def _d():
                        drain()

                def rloop(r4, c2, _bg=bg, _eg=eg, _sbs=sbs, _ses=ses,
                          _sds=sds, _sas=sas):
                    r0 = r4 * _RUR
                    for ur in range(_RUR):
                        rr = r0 + ur
                        rvec = jnp.full((16,), rr, dtype=jnp.int32)
                        for h in range(_DIM // _L):
                            sl = pl.ds(h * _L, _L)
                            bv = _bg[rr, sl]
                            ev = _eg[rr, sl]
                            t = bv + ev
                            a0 = bv + 2.0 * ev
                            plsc.store_scatter(_sbs, [kvs[h], rvec], bv)
                            plsc.store_scatter(_ses, [kvs[h], rvec], ev)
                            plsc.store_scatter(_sds, [kvs[h], rvec], _mod1(t))
                            plsc.store_scatter(_sas, [kvs[h], rvec],
                                               _mod1(a0))
                    return c2

                lax.fori_loop(0, _CHUNK // _RUR, rloop, 0)

                j = cc // tcw
                t = cc % tcw
                for tr in range(n_ktiles):
                    q0 = ((j * n_ktiles + tr) * n_tc + wid * tcw + t) * 8
                    dst = pl.ds(q0, 8)
                    for src_ref, out_ref in ((sbs, ob), (ses, oe),
                                             (sds, od), (sas, oa)):
                        pltpu.async_copy(
                            src_ref.at[pl.ds(tr * 8, 8), pl.ds(0, _CHUNK)],
                            out_ref.at[dst], wsem[s])
            return carry

        lax.fori_loop(0, n_groups, group, 0)

        for s in range(2):
            for o_ref in stag[s]:
                for tr in range(n_ktiles):
                    pltpu.make_async_copy(
                        o_ref.at[pl.ds(tr * 8, 8), pl.ds(0, _CHUNK)],
                        ob.at[pl.ds(0, 8)], wsem[s]).wait()

    return sc_kernel


def kernel(x, b_table, e_table):
    info = plsc.get_sparse_core_info()
    nw = info.num_cores * info.num_subcores
    nc = info.num_cores
    n_i, n_j = x.shape
    ipw = n_i // nw
    tcw = ipw // _CHUNK
    xq = (x.T.astype(jnp.int32)
          .reshape(n_j, nw, tcw, _CHUNK)
          .transpose(1, 0, 2, 3)
          .reshape(nw, n_j * tcw, _CHUNK))
    k = _make_sc_kernel(n_j, n_i, nw, nc)
    outs = k(xq, b_table, e_table)
    n_ktiles = _DIM // 8
    n_tc = n_i // _CHUNK
    res = []
    for o in outs:
        v5 = o.reshape(n_j, n_ktiles, n_tc, 8, _CHUNK)
        res.append(v5.transpose(2, 4, 0, 1, 3).reshape(n_i, n_j, _DIM))
    return tuple(res)


# speedup vs baseline: 1.0019x; 1.0019x over previous
import functools

import jax
import jax.numpy as jnp
from jax import lax
from jax.experimental import pallas as pl
from jax.experimental.pallas import tpu as pltpu
from jax.experimental.pallas import tpu_sc as plsc

_DIM = 32
_L = 16
_CHUNK = 128
_RING = 4
_PF = 3
_PITCH = _CHUNK + 1
_RUR = 4


def _mod1(v):
    r = lax.rem(v, 1.0)
    return jnp.where(r < 0.0, r + 1.0, r)


@functools.lru_cache(maxsize=None)
def _make_sc_kernel(n_j, n_i, nw, nc):
    ipw = n_i // nw
    tcw = ipw // _CHUNK
    n_chunks = n_j * tcw
    n_groups = n_chunks // _RING
    n_ktiles = _DIM // 8
    n_tc = n_i // _CHUNK
    out_flat = jax.ShapeDtypeStruct((n_j * n_ktiles * n_tc * 8, _CHUNK),
                                    jnp.float32)
    mesh = plsc.VectorSubcoreMesh(core_axis_name="c", subcore_axis_name="s")

    @functools.partial(
        pl.kernel,
        mesh=mesh,
        out_type=(out_flat,) * 4,
        compiler_params=pltpu.CompilerParams(
            use_tc_tiling_on_sc=False, needs_layout_passes=False),
        scratch_types=(
            [pltpu.VMEM((n_chunks, _CHUNK), jnp.int32)]
            + [pltpu.VMEM((_CHUNK, _DIM), jnp.float32)] * (2 * _RING)
            + [pltpu.VMEM((_DIM, _PITCH), jnp.float32)] * 8
            + [pltpu.SemaphoreType.DMA] * (_RING + 2)
        ),
    )
    def sc_kernel(xq, bt, et, ob, oe, od, oa, idx_v, *bufs):
        brefs = bufs[:_RING]
        erefs = bufs[_RING:2 * _RING]
        sb0, se0, sd0, sa0, sb1, se1, sd1, sa1 = bufs[2 * _RING:
                                                      2 * _RING + 8]
        gsem = bufs[2 * _RING + 8:3 * _RING + 8]
        wsem = bufs[3 * _RING + 8:]
        stag = ((sb0, se0, sd0, sa0), (sb1, se1, sd1, sa1))
        c_ax = lax.axis_index("c")
        s_ax = lax.axis_index("s")
        wid = s_ax * nc + c_ax
        pltpu.sync_copy(xq.at[wid], idx_v)

        kvs = [lax.iota(jnp.int32, 16) + (h * _L) for h in range(_DIM // _L)]

        def issue_gather(cidx, slot):
            pltpu.async_copy(bt.at[idx_v.at[cidx]], brefs[slot], gsem[slot])
            pltpu.async_copy(et.at[idx_v.at[cidx]], erefs[slot], gsem[slot])

        for u in range(_PF):
            issue_gather(u, u)

        def group(gi, carry):
            for u in range(_RING):
                cc = gi * _RING + u
                s = u & 1
                sbs, ses, sds, sas = stag[s]
                bg = brefs[u]
                eg = erefs[u]

                pslot = (u + _PF) % _RING
                if u == 0:
                    issue_gather(cc + _PF, pslot)
                else:
                    @pl.when(gi < n_groups - 1)
                    def _pref(_cc=cc, _ps=pslot):
                        issue_gather(_cc + _PF, _ps)

                pltpu.make_async_copy(
                    bt.at[idx_v.at[cc]], bg, gsem[u]).wait()
                pltpu.make_async_copy(
                    et.at[idx_v.at[cc]], eg, gsem[u]).wait()

                def drain(_s=s):
                    for o_ref in stag[_s]:
                        for tr in range(n_ktiles):
                            pltpu.make_async_copy(
                                o_ref.at[pl.ds(tr * 8, 8), pl.ds(0, _CHUNK)],
                                ob.at[pl.ds(0, 8)], wsem[_s]).wait()
                if u >= 2:
                    drain()
                else:
                    @pl.when(gi >= 1)
                    def _d():
                        drain()

                def rloop(r4, c2, _bg=bg, _eg=eg, _sbs=sbs, _ses=ses,
                          _sds=sds, _sas=sas):
                    r0 = r4 * _RUR
                    for ur in range(_RUR):
                        rr = r0 + ur
                        rvec = jnp.full((16,), rr, dtype=jnp.int32)
                        for h in range(_DIM // _L):
                            sl = pl.ds(h * _L, _L)
                            bv = _bg[rr, sl]
                            ev = _eg[rr, sl]
                            t = bv + ev
                            a0 = bv + 2.0 * ev
                            plsc.store_scatter(_sbs, [kvs[h], rvec], bv)
                            plsc.store_scatter(_ses, [kvs[h], rvec], ev)
                            plsc.store_scatter(_sds, [kvs[h], rvec], _mod1(t))
                            plsc.store_scatter(_sas, [kvs[h], rvec],
                                               _mod1(a0))
                    return c2

                lax.fori_loop(0, _CHUNK // _RUR, rloop, 0)

                j = cc // tcw
                t = cc % tcw
                for tr in range(n_ktiles):
                    q0 = ((j * n_ktiles + tr) * n_tc + wid * tcw + t) * 8
                    dst = pl.ds(q0, 8)
                    for src_ref, out_ref in ((sbs, ob), (ses, oe),
                                             (sds, od), (sas, oa)):
                        pltpu.async_copy(
                            src_ref.at[pl.ds(tr * 8, 8), pl.ds(0, _CHUNK)],
                            out_ref.at[dst], wsem[s])
            return carry

        lax.fori_loop(0, n_groups, group, 0)

        for s in range(2):
            for o_ref in stag[s]:
                for tr in range(n_ktiles):
                    pltpu.make_async_copy(
                        o_ref.at[pl.ds(tr * 8, 8), pl.ds(0, _CHUNK)],
                        ob.at[pl.ds(0, 8)], wsem[s]).wait()

    return sc_kernel


def kernel(x, b_table, e_table):
    info = plsc.get_sparse_core_info()
    nw = info.num_cores * info.num_subcores
    nc = info.num_cores
    n_i, n_j = x.shape
    ipw = n_i // nw
    tcw = ipw // _CHUNK
    xq = (x.T.astype(jnp.int32)
          .reshape(n_j, nw, tcw, _CHUNK)
          .transpose(1, 0, 2, 3)
          .reshape(nw, n_j * tcw, _CHUNK))
    k = _make_sc_kernel(n_j, n_i, nw, nc)
    outs = k(xq, b_table, e_table)
    n_ktiles = _DIM // 8
    n_tc = n_i // _CHUNK
    res = []
    for o in outs:
        v5 = o.reshape(n_j, n_ktiles, n_tc, 8, _CHUNK)
        res.append(v5.transpose(2, 4, 0, 1, 3).reshape(n_i, n_j, _DIM))
    return tuple(res)
